# baseline (speedup 1.0000x reference)
import functools

import jax
import jax.numpy as jnp
from jax import lax
from jax.experimental import pallas as pl
from jax.experimental.pallas import tpu as pltpu
from jax.experimental.pallas import tpu_sc as plsc

C = 128
N = 10000
K = 16
OUT = 128

NW = 32
PW = 320
NP = NW * PW
CH = 8
CE = CH * K
NCH = PW // CH
NV = OUT // 16


def _mm_body(x_ref, w_ref, at_ref, bt_ref, sa_ref, sa2_ref):
    X = x_ref[...]
    W = w_ref[...]
    W1 = W[:, :C]
    W2 = W[:, C:]
    At = lax.dot_general(X, W1 + W2, (((0,), (1,)), ((), ())),
                         preferred_element_type=jnp.float32)
    Bt = lax.dot_general(X, W2, (((0,), (1,)), ((), ())),
                         preferred_element_type=jnp.float32)
    at_ref[pl.ds(0, N), :] = At
    at_ref[pl.ds(N, NP - N), :] = jnp.zeros((NP - N, OUT), jnp.float32)
    bt_ref[pl.ds(0, N), :] = Bt
    bt_ref[pl.ds(N, NP - N), :] = jnp.zeros((NP - N, OUT), jnp.float32)
    sa_ref[...] = jnp.sum(At, axis=0, keepdims=True)
    sa2_ref[...] = jnp.sum(At * At, axis=0, keepdims=True)


def _matmuls(x2d, W):
    return pl.pallas_call(
        _mm_body,
        out_shape=[
            jax.ShapeDtypeStruct((NP, OUT), jnp.float32),
            jax.ShapeDtypeStruct((NP, OUT), jnp.float32),
            jax.ShapeDtypeStruct((1, OUT), jnp.float32),
            jax.ShapeDtypeStruct((1, OUT), jnp.float32),
        ],
    )(x2d, W)


NB = 4


def _sc_body(table_hbm, ap_hbm, idx_hbm, pre_hbm, s1_hbm, cnt_hbm, as_hbm,
             idx_v, rows0, rows1, rows2, rows3, a0, a1, a2, a3,
             pb0, pb1, pb2, pb3, cnt_v, p1_v, p3_v,
             gsem0, gsem1, gsem2, gsem3, asem0, asem1, asem2, asem3,
             psem0, psem1, psem2, psem3):
    sid = lax.axis_index("s")
    wid = sid * 2 + lax.axis_index("c")
    nbase = wid * PW
    pltpu.sync_copy(idx_hbm.at[wid], idx_v)

    rows = (rows0, rows1, rows2, rows3)
    abufs = (a0, a1, a2, a3)
    pbufs = (pb0, pb1, pb2, pb3)
    gsems = (gsem0, gsem1, gsem2, gsem3)
    asems = (asem0, asem1, asem2, asem3)
    psems = (psem0, psem1, psem2, psem3)

    def gcopy(c, b):
        return pltpu.make_async_copy(
            table_hbm.at[idx_v.at[c]], rows[b], gsems[b])

    def acopy(c, b):
        return pltpu.make_async_copy(
            ap_hbm.at[pl.ds(nbase + c * CH, CH)], abufs[b], asems[b])

    def pcopy(c, b):
        return pltpu.make_async_copy(
            pbufs[b], pre_hbm.at[pl.ds(nbase + c * CH, CH)], psems[b])

    for b in range(NB):
        gcopy(b, b).start()
        acopy(b, b).start()

    zero = jnp.zeros((16,), jnp.float32)

    def zbody(i, _):
        cnt_v[pl.ds(i * 16, 16)] = zero
        return 0

    lax.fori_loop(0, NP // 16, zbody, 0)

    one = jnp.ones((16,), jnp.float32)
    carry0 = (zero,) * (2 * NV)

    def chunk4(c4, carry):
        for b in range(NB):
            c = c4 * NB + b
            gcopy(c, b).wait()
            acopy(c, b).wait()
            @pl.when(c4 > 0)
            def _():
                pcopy(c - NB, b).wait()
            rv = rows[b]
            av = abufs[b]
            pv = pbufs[b]

            def hist(g, _):
                ix = idx_v[c, pl.ds(g * 16, 16)]
                plsc.addupdate_scatter(cnt_v, [ix], one)
                return 0

            lax.fori_loop(0, CH, hist, 0)

            def node(i, cr):
                r0 = i * K
                out = []
                for v in range(NV):
                    sl = pl.ds(v * 16, 16)
                    m = rv[r0, sl]
                    s = m
                    for r in range(1, K):
                        t = rv[r0 + r, sl]
                        m = jnp.minimum(m, t)
                        s = s + t
                    a = av[i, sl]
                    pv[i, sl] = a - m
                    out.append(cr[v] + s)
                    out.append(cr[NV + v] + a * s)
                return tuple(out[0::2]) + tuple(out[1::2])

            carry = lax.fori_loop(0, CH, node, carry)
            pcopy(c, b).start()

            @pl.when(c4 < NCH // NB - 1)
            def _():
                gcopy(c + NB, b).start()
                acopy(c + NB, b).start()
        return carry

    carry = lax.fori_loop(0, NCH // NB, chunk4, carry0)
    for b in range(NB):
        pcopy(NCH - NB + b, b).wait()

    for v in range(NV):
        sl = pl.ds(v * 16, 16)
        p1_v[0, sl] = carry[v]
        p3_v[0, sl] = carry[NV + v]
    pltpu.sync_copy(p1_v, s1_hbm.at[pl.ds(wid, 1)])
    pltpu.sync_copy(p3_v, as_hbm.at[pl.ds(wid, 1)])
    pltpu.sync_copy(cnt_v, cnt_hbm.at[wid])


def _sc_gather(bt, at, idx3):
    mesh = plsc.VectorSubcoreMesh(core_axis_name="c", subcore_axis_name="s")
    f = functools.partial(
        pl.kernel,
        out_type=[
            jax.ShapeDtypeStruct((NP, OUT), jnp.float32),
            jax.ShapeDtypeStruct((NW, OUT), jnp.float32),
            jax.ShapeDtypeStruct((NW, NP), jnp.float32),
            jax.ShapeDtypeStruct((NW, OUT), jnp.float32),
        ],
        mesh=mesh,
        compiler_params=pltpu.CompilerParams(needs_layout_passes=False),
        scratch_types=(
            [pltpu.VMEM((NCH, CE), jnp.int32)]
            + [pltpu.VMEM((CE, OUT), jnp.float32)] * NB
            + [pltpu.VMEM((CH, OUT), jnp.float32)] * NB
            + [pltpu.VMEM((CH, OUT), jnp.float32)] * NB
            + [pltpu.VMEM((NP,), jnp.float32)]
            + [pltpu.VMEM((1, OUT), jnp.float32)] * 2
            + [pltpu.SemaphoreType.DMA] * (3 * NB)
        ),
    )(_sc_body)
    return f(bt, at, idx3)


def _fin_body(pre_ref, s1_ref, cnt_ref, as_ref, sa_ref, sa2_ref, bt_ref,
              g_ref, b_ref, o_ref):
    inv_nk = 1.0 / (N * K)
    sS1 = jnp.sum(s1_ref[...], axis=0, keepdims=True)
    sAS1 = jnp.sum(as_ref[...], axis=0, keepdims=True)
    cnt = jnp.sum(cnt_ref[...], axis=0, keepdims=True)
    Bt = bt_ref[...]
    sS2 = lax.dot_general(cnt, Bt * Bt, (((1,), (0,)), ((), ())),
                          preferred_element_type=jnp.float32)
    mean = (K * sa_ref[...] - sS1) * inv_nk
    e2 = (K * sa2_ref[...] - 2.0 * sAS1 + sS2) * inv_nk
    var = e2 - mean * mean
    inv = lax.rsqrt(var + 1e-5)
    pre = pre_ref[pl.ds(0, N), :]
    h = (pre - mean) * (inv * g_ref[...]) + b_ref[...]
    o_ref[...] = jnp.where(h >= 0, h, 0.2 * h)


def _finalize(pre, s1p, cntp, asp, sa, sa2, bt, gamma, beta):
    return pl.pallas_call(
        _fin_body,
        out_shape=jax.ShapeDtypeStruct((N, OUT), jnp.float32),
    )(pre, s1p, cntp, asp, sa, sa2, bt,
      gamma.reshape(1, OUT), beta.reshape(1, OUT))


def kernel(x, edge_index, W, gamma, beta):
    x2d = x.reshape(C, N)
    at, bt, sa, sa2 = _matmuls(x2d, W)
    idx_flat = edge_index.reshape(N * K)
    npad = NP * K - N * K
    pad_vals = N + (jnp.arange(npad, dtype=jnp.int32) % (NP - N))
    idx_pad = jnp.concatenate([idx_flat, pad_vals])
    idx3 = idx_pad.reshape(NW, NCH, CE)
    pre, s1p, cntp, asp = _sc_gather(bt, at, idx3)
    res = _finalize(pre, s1p, cntp, asp, sa, sa2, bt, gamma, beta)
    return jnp.transpose(res).reshape(1, OUT, N, 1)


# speedup vs baseline: 1.2384x; 1.2384x over previous
import functools

import jax
import jax.numpy as jnp
from jax import lax
from jax.experimental import pallas as pl
from jax.experimental.pallas import tpu as pltpu
from jax.experimental.pallas import tpu_sc as plsc

C = 128
N = 10000
K = 16
OUT = 128

NW = 32
PW = 320
NP = NW * PW
CH = 8
CE = CH * K
NCH = PW // CH
NV = OUT // 16


def _mm_body(x_ref, w_ref, at_ref, bt_ref, sa_ref, sa2_ref):
    X = x_ref[...]
    W = w_ref[...]
    W1 = W[:, :C]
    W2 = W[:, C:]
    At = lax.dot_general(X, W1 + W2, (((0,), (1,)), ((), ())),
                         preferred_element_type=jnp.float32)
    Bt = lax.dot_general(X, W2, (((0,), (1,)), ((), ())),
                         preferred_element_type=jnp.float32)
    at_ref[pl.ds(0, N), :] = At
    at_ref[pl.ds(N, NP - N), :] = jnp.zeros((NP - N, OUT), jnp.float32)
    bt_ref[pl.ds(0, N), :] = Bt
    bt_ref[pl.ds(N, NP - N), :] = jnp.zeros((NP - N, OUT), jnp.float32)
    sa_ref[...] = jnp.sum(At, axis=0, keepdims=True)
    sa2_ref[...] = jnp.sum(At * At, axis=0, keepdims=True)


def _matmuls(x2d, W):
    return pl.pallas_call(
        _mm_body,
        out_shape=[
            jax.ShapeDtypeStruct((NP, OUT), jnp.float32),
            jax.ShapeDtypeStruct((NP, OUT), jnp.float32),
            jax.ShapeDtypeStruct((1, OUT), jnp.float32),
            jax.ShapeDtypeStruct((1, OUT), jnp.float32),
        ],
    )(x2d, W)


NB = 4


def _sc_body(table_hbm, ap_hbm, idx_hbm, pre_hbm, s1_hbm, s2_hbm, as_hbm,
             idx_v, rows0, rows1, rows2, rows3, a_v, pb0, pb1, pb2, pb3,
             p1_v, p2_v, p3_v,
             gsem0, gsem1, gsem2, gsem3, asem,
             psem0, psem1, psem2, psem3):
    sid = lax.axis_index("s")
    wid = sid * 2 + lax.axis_index("c")
    nbase = wid * PW
    pltpu.sync_copy(idx_hbm.at[wid], idx_v)
    acp = pltpu.make_async_copy(ap_hbm.at[pl.ds(nbase, PW)], a_v, asem)
    acp.start()

    rows = (rows0, rows1, rows2, rows3)
    pbufs = (pb0, pb1, pb2, pb3)
    gsems = (gsem0, gsem1, gsem2, gsem3)
    psems = (psem0, psem1, psem2, psem3)

    def gcopy(c, b):
        return pltpu.make_async_copy(
            table_hbm.at[idx_v.at[c]], rows[b], gsems[b])

    def pcopy(c, b):
        return pltpu.make_async_copy(
            pbufs[b], pre_hbm.at[pl.ds(nbase + c * CH, CH)], psems[b])

    for b in range(NB):
        gcopy(b, b).start()
    acp.wait()

    zero = jnp.zeros((16,), jnp.float32)
    carry0 = (zero,) * (3 * NV)

    def chunk4(c4, carry):
        for b in range(NB):
            c = c4 * NB + b
            gcopy(c, b).wait()
            @pl.when(c4 > 0)
            def _():
                pcopy(c - NB, b).wait()
            rv = rows[b]
            pv = pbufs[b]

            def node(i, cr):
                r0 = i * K
                out = []
                for v in range(NV):
                    sl = pl.ds(v * 16, 16)
                    m = rv[r0, sl]
                    s = m
                    q = m * m
                    for r in range(1, K):
                        t = rv[r0 + r, sl]
                        m = jnp.minimum(m, t)
                        s = s + t
                        q = q + t * t
                    a = a_v[c * CH + i, sl]
                    pv[i, sl] = a - m
                    out.append(cr[v] + s)
                    out.append(cr[NV + v] + q)
                    out.append(cr[2 * NV + v] + a * s)
                return tuple(out[0::3]) + tuple(out[1::3]) + tuple(out[2::3])

            carry = lax.fori_loop(0, CH, node, carry)
            pcopy(c, b).start()

            @pl.when(c4 < NCH // NB - 1)
            def _():
                gcopy(c + NB, b).start()
        return carry

    carry = lax.fori_loop(0, NCH // NB, chunk4, carry0)
    for b in range(NB):
        pcopy(NCH - NB + b, b).wait()

    for v in range(NV):
        sl = pl.ds(v * 16, 16)
        p1_v[0, sl] = carry[v]
        p2_v[0, sl] = carry[NV + v]
        p3_v[0, sl] = carry[2 * NV + v]
    pltpu.sync_copy(p1_v, s1_hbm.at[pl.ds(wid, 1)])
    pltpu.sync_copy(p2_v, s2_hbm.at[pl.ds(wid, 1)])
    pltpu.sync_copy(p3_v, as_hbm.at[pl.ds(wid, 1)])


def _sc_gather(bt, at, idx3):
    mesh = plsc.VectorSubcoreMesh(core_axis_name="c", subcore_axis_name="s")
    f = functools.partial(
        pl.kernel,
        out_type=[
            jax.ShapeDtypeStruct((NP, OUT), jnp.float32),
            jax.ShapeDtypeStruct((NW, OUT), jnp.float32),
            jax.ShapeDtypeStruct((NW, OUT), jnp.float32),
            jax.ShapeDtypeStruct((NW, OUT), jnp.float32),
        ],
        mesh=mesh,
        scratch_types=(
            [pltpu.VMEM((NCH, CE), jnp.int32)]
            + [pltpu.VMEM((CE, OUT), jnp.float32)] * NB
            + [pltpu.VMEM((PW, OUT), jnp.float32)]
            + [pltpu.VMEM((CH, OUT), jnp.float32)] * NB
            + [pltpu.VMEM((1, OUT), jnp.float32)] * 3
            + [pltpu.SemaphoreType.DMA] * (2 * NB + 1)
        ),
    )(_sc_body)
    return f(bt, at, idx3)


def _fin_body(pre_ref, s1_ref, s2_ref, as_ref, sa_ref, sa2_ref,
              g_ref, b_ref, o_ref):
    inv_nk = 1.0 / (N * K)
    sS1 = jnp.sum(s1_ref[...], axis=0, keepdims=True)
    sS2 = jnp.sum(s2_ref[...], axis=0, keepdims=True)
    sAS1 = jnp.sum(as_ref[...], axis=0, keepdims=True)
    mean = (K * sa_ref[...] - sS1) * inv_nk
    e2 = (K * sa2_ref[...] - 2.0 * sAS1 + sS2) * inv_nk
    var = e2 - mean * mean
    inv = lax.rsqrt(var + 1e-5)
    pre = pre_ref[pl.ds(0, N), :]
    h = (pre - mean) * (inv * g_ref[...]) + b_ref[...]
    o_ref[...] = jnp.where(h >= 0, h, 0.2 * h)


def _finalize(pre, s1p, s2p, asp, sa, sa2, gamma, beta):
    return pl.pallas_call(
        _fin_body,
        out_shape=jax.ShapeDtypeStruct((N, OUT), jnp.float32),
    )(pre, s1p, s2p, asp, sa, sa2,
      gamma.reshape(1, OUT), beta.reshape(1, OUT))


def kernel(x, edge_index, W, gamma, beta):
    x2d = x.reshape(C, N)
    at, bt, sa, sa2 = _matmuls(x2d, W)
    idx_flat = edge_index.reshape(N * K)
    npad = NP * K - N * K
    pad_vals = N + (jnp.arange(npad, dtype=jnp.int32) % (NP - N))
    idx_pad = jnp.concatenate([idx_flat, pad_vals])
    idx3 = idx_pad.reshape(NW, NCH, CE)
    pre, s1p, s2p, asp = _sc_gather(bt, at, idx3)
    res = _finalize(pre, s1p, s2p, asp, sa, sa2, gamma, beta)
    return jnp.transpose(res).reshape(1, OUT, N, 1)
